# Initial kernel scaffold; baseline (speedup 1.0000x reference)
import jax
import jax.numpy as jnp
from jax.experimental import pallas as pl


def kernel(x, emb_table, global_pos_embed, global_pad):
    raise NotImplementedError("write your pallas kernel here")


# speedup vs baseline: 3.6913x; 3.6913x over previous
import functools

import jax
import jax.numpy as jnp
from jax import lax
from jax.experimental import pallas as pl
from jax.experimental.pallas import tpu as pltpu
from jax.experimental.pallas import tpu_sc as plsc

_PATCH = 4
_D = 32
_EMB_ROWS = 256
_TROWS = 264
_CHUNK = 128


def _fuse_body(emb_ref, pos_ref, pad_ref, o_ref):
    row = lax.broadcasted_iota(jnp.int32, (_TROWS, _D), 0)
    pos = pos_ref[...]
    o_ref[...] = emb_ref[...] + pad_ref[...] + jnp.where(
        row < _EMB_ROWS, pos, jnp.zeros_like(pos)
    )


def _build_fused_table(emb_table, global_pos_embed, global_pad):
    emb264 = jnp.pad(emb_table, ((0, _TROWS - _EMB_ROWS), (0, 0)))
    pad264 = jnp.pad(
        global_pad.reshape(_PATCH, _D),
        ((_EMB_ROWS, _TROWS - _EMB_ROWS - _PATCH), (0, 0)),
    )
    pos2d = global_pos_embed.reshape(1, _D)
    return pl.pallas_call(
        _fuse_body,
        out_shape=jax.ShapeDtypeStruct((_TROWS, _D), jnp.float32),
    )(emb264, pos2d, pad264)


def _make_gather(nw, n_chunks):
    mesh = plsc.VectorSubcoreMesh(core_axis_name="c", subcore_axis_name="s")
    info = plsc.get_sparse_core_info()
    nc = info.num_cores
    rows_per_w = n_chunks * _CHUNK

    @functools.partial(
        pl.kernel,
        mesh=mesh,
        compiler_params=pltpu.CompilerParams(use_tc_tiling_on_sc=False),
        out_type=jax.ShapeDtypeStruct((nw, rows_per_w, _D), jnp.float32),
        scratch_types=[
            pltpu.VMEM((n_chunks, _CHUNK), jnp.int32),
            pltpu.VMEM((rows_per_w, _D), jnp.float32),
            pltpu.SemaphoreType.DMA,
        ],
    )
    def gather(table_hbm, idx_hbm, out_hbm, idx_v, rows_v, sem):
        wid = lax.axis_index("s") * nc + lax.axis_index("c")
        pltpu.sync_copy(idx_hbm.at[wid], idx_v)
        copies = [
            pltpu.async_copy(
                table_hbm.at[idx_v.at[j]],
                rows_v.at[pl.ds(j * _CHUNK, _CHUNK)],
                sem,
            )
            for j in range(n_chunks)
        ]
        for cp in copies:
            cp.wait()
        pltpu.sync_copy(rows_v, out_hbm.at[wid])

    return gather


def kernel(x, emb_table, global_pos_embed, global_pad):
    B, S = x.shape
    assert S % _PATCH == 0
    n_rows = B * S
    info = plsc.get_sparse_core_info()
    nw = info.num_cores * info.num_subcores
    assert n_rows % (nw * _CHUNK) == 0
    n_chunks = n_rows // (nw * _CHUNK)

    fused = _build_fused_table(emb_table, global_pos_embed, global_pad)

    x32 = x.astype(jnp.int32)
    pad_idx = jnp.arange(_EMB_ROWS, _EMB_ROWS + _PATCH, dtype=jnp.int32)
    idx = jnp.concatenate(
        [jnp.broadcast_to(pad_idx[None, :], (B, _PATCH)), x32[:, : S - _PATCH]],
        axis=1,
    ).reshape(nw, n_chunks, _CHUNK)

    out = _make_gather(nw, n_chunks)(fused, idx)
    return out.reshape(B, S // _PATCH, _PATCH * _D)
